# initial kernel scaffold (unmeasured)
import jax
import jax.numpy as jnp
from jax import lax
from jax.experimental import pallas as pl
from jax.experimental.pallas import tpu as pltpu

M = 2048
D = 2048
F_SHARD = 8192
F_BLK = 512
N_Y = 4


def _matmul_body(dy_ref, w_ref, out_ref, acc_ref):
    k = pl.program_id(0)

    @pl.when(k == 0)
    def _():
        acc_ref[...] = jnp.zeros_like(acc_ref)

    a = dy_ref[...].astype(jnp.bfloat16)
    b = w_ref[...].astype(jnp.bfloat16)
    acc_ref[...] += lax.dot_general(
        a, b, (((1,), (1,)), ((), ())), preferred_element_type=jnp.float32
    )

    @pl.when(k == pl.num_programs(0) - 1)
    def _():
        out_ref[...] = acc_ref[...].astype(jnp.bfloat16)


def _partial_matmul(dy, w):
    return pl.pallas_call(
        _matmul_body,
        grid=(F_SHARD // F_BLK,),
        in_specs=[
            pl.BlockSpec((M, F_BLK), lambda k: (0, k)),
            pl.BlockSpec((D, F_BLK), lambda k: (0, k)),
        ],
        out_specs=pl.BlockSpec((M, D), lambda k: (0, 0)),
        out_shape=jax.ShapeDtypeStruct((M, D), jnp.bfloat16),
        scratch_shapes=[pltpu.VMEM((M, D), jnp.float32)],
    )(dy, w)


def _ar_body(p_ref, out_ref, comm_ref, send_sem, recv_sems):
    my_x = lax.axis_index("x")
    my_y = lax.axis_index("y")
    my_z = lax.axis_index("z")
    right = (my_y + 1) % N_Y
    left = (my_y - 1) % N_Y

    barrier = pltpu.get_barrier_semaphore()
    for nbr in (left, right):
        pl.semaphore_signal(
            barrier, inc=1, device_id=(my_x, nbr, my_z),
            device_id_type=pl.DeviceIdType.MESH,
        )
    pl.semaphore_wait(barrier, 2)

    out_ref[...] = p_ref[...].astype(jnp.float32)
    comm_ref[0] = p_ref[...]

    for h in range(N_Y - 1):
        rdma = pltpu.make_async_remote_copy(
            src_ref=comm_ref.at[h],
            dst_ref=comm_ref.at[h + 1],
            send_sem=send_sem,
            recv_sem=recv_sems.at[h],
            device_id=(my_x, right, my_z),
            device_id_type=pl.DeviceIdType.MESH,
        )
        rdma.start()
        rdma.wait()
        out_ref[...] += comm_ref[h + 1].astype(jnp.float32)


def _all_reduce_y(p):
    return pl.pallas_call(
        _ar_body,
        out_shape=jax.ShapeDtypeStruct((M, D), jnp.float32),
        in_specs=[pl.BlockSpec(memory_space=pltpu.VMEM)],
        out_specs=pl.BlockSpec(memory_space=pltpu.VMEM),
        scratch_shapes=[
            pltpu.VMEM((N_Y, M, D), jnp.bfloat16),
            pltpu.SemaphoreType.DMA,
            pltpu.SemaphoreType.DMA((N_Y - 1,)),
        ],
        compiler_params=pltpu.CompilerParams(collective_id=0),
    )(p)


def kernel(dy, W):
    p = _partial_matmul(dy, W)
    return _all_reduce_y(p)


# baseline (device time: 393094 ns/iter reference)
import jax
import jax.numpy as jnp
from jax import lax
from jax.experimental import pallas as pl
from jax.experimental.pallas import tpu as pltpu

M = 2048
D = 2048
F_SHARD = 8192
F_BLK = 512
N_Y = 4


def _matmul_body(dy_ref, w_ref, out_ref, acc_ref):
    k = pl.program_id(0)

    @pl.when(k == 0)
    def _():
        acc_ref[...] = jnp.zeros_like(acc_ref)

    a = dy_ref[...].astype(jnp.bfloat16)
    b = w_ref[...].astype(jnp.bfloat16)
    acc_ref[...] += lax.dot_general(
        a, b, (((1,), (1,)), ((), ())), preferred_element_type=jnp.float32
    )

    @pl.when(k == pl.num_programs(0) - 1)
    def _():
        out_ref[...] = acc_ref[...].astype(jnp.bfloat16)


def _partial_matmul(dy, w):
    return pl.pallas_call(
        _matmul_body,
        grid=(F_SHARD // F_BLK,),
        in_specs=[
            pl.BlockSpec((M, F_BLK), lambda k: (0, k)),
            pl.BlockSpec((D, F_BLK), lambda k: (0, k)),
        ],
        out_specs=pl.BlockSpec((M, D), lambda k: (0, 0)),
        out_shape=jax.ShapeDtypeStruct((M, D), jnp.bfloat16),
        scratch_shapes=[pltpu.VMEM((M, D), jnp.float32)],
        compiler_params=pltpu.CompilerParams(
            vmem_limit_bytes=100 * 1024 * 1024,
        ),
    )(dy, w)


def _ar_body(p_ref, out_ref, comm_ref, send_sem, recv_sems):
    my_x = lax.axis_index("x")
    my_y = lax.axis_index("y")
    my_z = lax.axis_index("z")
    right = (my_y + 1) % N_Y
    left = (my_y - 1) % N_Y

    barrier = pltpu.get_barrier_semaphore()
    for nbr in (left, right):
        pl.semaphore_signal(
            barrier, inc=1, device_id=(my_x, nbr, my_z),
            device_id_type=pl.DeviceIdType.MESH,
        )
    pl.semaphore_wait(barrier, 2)

    out_ref[...] = p_ref[...].astype(jnp.float32)
    comm_ref[0] = p_ref[...]

    for h in range(N_Y - 1):
        rdma = pltpu.make_async_remote_copy(
            src_ref=comm_ref.at[h],
            dst_ref=comm_ref.at[h + 1],
            send_sem=send_sem,
            recv_sem=recv_sems.at[h],
            device_id=(my_x, right, my_z),
            device_id_type=pl.DeviceIdType.MESH,
        )
        rdma.start()
        rdma.wait()
        out_ref[...] += comm_ref[h + 1].astype(jnp.float32)


def _all_reduce_y(p):
    return pl.pallas_call(
        _ar_body,
        out_shape=jax.ShapeDtypeStruct((M, D), jnp.float32),
        in_specs=[pl.BlockSpec(memory_space=pltpu.VMEM)],
        out_specs=pl.BlockSpec(memory_space=pltpu.VMEM),
        scratch_shapes=[
            pltpu.VMEM((N_Y, M, D), jnp.bfloat16),
            pltpu.SemaphoreType.DMA,
            pltpu.SemaphoreType.DMA((N_Y - 1,)),
        ],
        compiler_params=pltpu.CompilerParams(
            collective_id=0,
            vmem_limit_bytes=100 * 1024 * 1024,
        ),
    )(p)


def kernel(dy, W):
    p = _partial_matmul(dy, W)
    return _all_reduce_y(p)


# device time: 153414 ns/iter; 2.5623x vs baseline; 2.5623x over previous
import jax
import jax.numpy as jnp
from jax import lax
from jax.experimental import pallas as pl
from jax.experimental.pallas import tpu as pltpu

M = 2048
D = 2048
F_SHARD = 8192
F_BLK = 512
N_Y = 4
N_RING = 8
CH = M // N_RING


def _matmul_body(dy_ref, w_ref, out_ref, acc_ref):
    k = pl.program_id(0)

    @pl.when(k == 0)
    def _():
        acc_ref[...] = jnp.zeros_like(acc_ref)

    a = dy_ref[...].astype(jnp.bfloat16)
    b = w_ref[...].astype(jnp.bfloat16)
    acc_ref[...] += lax.dot_general(
        a, b, (((1,), (1,)), ((), ())), preferred_element_type=jnp.float32
    )

    @pl.when(k == pl.num_programs(0) - 1)
    def _():
        out_ref[...] = acc_ref[...].astype(jnp.bfloat16)


def _partial_matmul(dy_slice, w):
    return pl.pallas_call(
        _matmul_body,
        grid=(F_SHARD // F_BLK,),
        in_specs=[
            pl.BlockSpec((CH, F_BLK), lambda k: (0, k)),
            pl.BlockSpec((D, F_BLK), lambda k: (0, k)),
        ],
        out_specs=pl.BlockSpec((CH, D), lambda k: (0, 0)),
        out_shape=jax.ShapeDtypeStruct((CH, D), jnp.bfloat16),
        scratch_shapes=[pltpu.VMEM((CH, D), jnp.float32)],
        compiler_params=pltpu.CompilerParams(
            vmem_limit_bytes=100 * 1024 * 1024,
        ),
    )(dy_slice, w)


def _ar_body(p_ref, out_ref, comm_ref, ag_ref, acc_ref,
             yar_send, yar_recvs, cw_send, ccw_send, cw_recvs, ccw_recvs):
    my_x = lax.axis_index("x")
    my_y = lax.axis_index("y")
    my_z = lax.axis_index("z")
    y_right = (my_y + 1) % N_Y
    y_left = (my_y - 1) % N_Y

    r = jnp.where(my_x == 0, my_z, 7 - my_z)
    cw_x = jnp.where(my_x == 0,
                     jnp.where(my_z == 3, 1, 0),
                     jnp.where(my_z == 0, 0, 1))
    cw_z = jnp.where(my_x == 0,
                     jnp.where(my_z == 3, 3, my_z + 1),
                     jnp.where(my_z == 0, 0, my_z - 1))
    ccw_x = jnp.where(my_x == 0,
                      jnp.where(my_z == 0, 1, 0),
                      jnp.where(my_z == 3, 0, 1))
    ccw_z = jnp.where(my_x == 0,
                      jnp.where(my_z == 0, 0, my_z - 1),
                      jnp.where(my_z == 3, 3, my_z + 1))

    barrier = pltpu.get_barrier_semaphore()
    for dev in ((my_x, y_left, my_z), (my_x, y_right, my_z),
                (cw_x, my_y, cw_z), (ccw_x, my_y, ccw_z)):
        pl.semaphore_signal(
            barrier, inc=1, device_id=dev,
            device_id_type=pl.DeviceIdType.MESH,
        )
    pl.semaphore_wait(barrier, 4)

    acc_ref[...] = p_ref[...].astype(jnp.float32)
    comm_ref[0] = p_ref[...]
    for h in range(N_Y - 1):
        rdma = pltpu.make_async_remote_copy(
            src_ref=comm_ref.at[h],
            dst_ref=comm_ref.at[h + 1],
            send_sem=yar_send,
            recv_sem=yar_recvs.at[h],
            device_id=(my_x, y_right, my_z),
            device_id_type=pl.DeviceIdType.MESH,
        )
        rdma.start()
        rdma.wait()
        acc_ref[...] += comm_ref[h + 1].astype(jnp.float32)

    ag_ref[r] = acc_ref[...].astype(jnp.bfloat16)
    out_ref[pl.ds(r * CH, CH), :] = acc_ref[...]

    def mk_cw(s):
        return pltpu.make_async_remote_copy(
            src_ref=ag_ref.at[(r - s) % N_RING],
            dst_ref=ag_ref.at[(r - s) % N_RING],
            send_sem=cw_send,
            recv_sem=cw_recvs.at[s],
            device_id=(cw_x, my_y, cw_z),
            device_id_type=pl.DeviceIdType.MESH,
        )

    def mk_ccw(s):
        return pltpu.make_async_remote_copy(
            src_ref=ag_ref.at[(r + s) % N_RING],
            dst_ref=ag_ref.at[(r + s) % N_RING],
            send_sem=ccw_send,
            recv_sem=ccw_recvs.at[s],
            device_id=(ccw_x, my_y, ccw_z),
            device_id_type=pl.DeviceIdType.MESH,
        )

    rd_cw = mk_cw(0)
    rd_cw.start()
    rd_ccw = mk_ccw(0)
    rd_ccw.start()
    for s in range(4):
        rd_cw.wait()
        if s < 3:
            rd_ccw.wait()
        nxt_cw = nxt_ccw = None
        if s + 1 < 4:
            nxt_cw = mk_cw(s + 1)
            nxt_cw.start()
        if s + 1 < 3:
            nxt_ccw = mk_ccw(s + 1)
            nxt_ccw.start()
        j = (r - 1 - s) % N_RING
        out_ref[pl.ds(j * CH, CH), :] = ag_ref[j].astype(jnp.float32)
        if s < 3:
            j2 = (r + 1 + s) % N_RING
            out_ref[pl.ds(j2 * CH, CH), :] = ag_ref[j2].astype(jnp.float32)
        rd_cw, rd_ccw = nxt_cw, nxt_ccw


def _allreduce_allgather(p):
    return pl.pallas_call(
        _ar_body,
        out_shape=jax.ShapeDtypeStruct((M, D), jnp.float32),
        in_specs=[pl.BlockSpec(memory_space=pltpu.VMEM)],
        out_specs=pl.BlockSpec(memory_space=pltpu.VMEM),
        scratch_shapes=[
            pltpu.VMEM((N_Y, CH, D), jnp.bfloat16),
            pltpu.VMEM((N_RING, CH, D), jnp.bfloat16),
            pltpu.VMEM((CH, D), jnp.float32),
            pltpu.SemaphoreType.DMA,
            pltpu.SemaphoreType.DMA((N_Y - 1,)),
            pltpu.SemaphoreType.DMA,
            pltpu.SemaphoreType.DMA,
            pltpu.SemaphoreType.DMA((4,)),
            pltpu.SemaphoreType.DMA((3,)),
        ],
        compiler_params=pltpu.CompilerParams(
            collective_id=0,
            vmem_limit_bytes=100 * 1024 * 1024,
        ),
    )(p)


def kernel(dy, W):
    my_x = lax.axis_index("x")
    my_z = lax.axis_index("z")
    r = jnp.where(my_x == 0, my_z, 7 - my_z)
    dy_slice = lax.dynamic_slice(dy, (r * CH, 0), (CH, F_SHARD))
    p = _partial_matmul(dy_slice, W)
    return _allreduce_allgather(p)


# device time: 137523 ns/iter; 2.8584x vs baseline; 1.1156x over previous
import jax
import jax.numpy as jnp
from jax import lax
from jax.experimental import pallas as pl
from jax.experimental.pallas import tpu as pltpu

M = 2048
D = 2048
HD = D // 2
F_SHARD = 8192
F_BLK = 512
N_Y = 4
N_RING = 8
CH = M // N_RING


def _matmul_body(r_ref, dy_ref, w_ref, out_ref, acc_ref):
    k = pl.program_id(0)

    @pl.when(k == 0)
    def _():
        acc_ref[...] = jnp.zeros_like(acc_ref)

    a = dy_ref[...].astype(jnp.bfloat16)
    b = w_ref[...].astype(jnp.bfloat16)
    acc_ref[...] += lax.dot_general(
        a, b, (((1,), (1,)), ((), ())), preferred_element_type=jnp.float32
    )

    @pl.when(k == pl.num_programs(0) - 1)
    def _():
        out_ref[...] = acc_ref[...].astype(jnp.bfloat16)


def _partial_matmul(r, dy, w):
    grid_spec = pltpu.PrefetchScalarGridSpec(
        num_scalar_prefetch=1,
        grid=(F_SHARD // F_BLK,),
        in_specs=[
            pl.BlockSpec((CH, F_BLK), lambda k, r_sc: (r_sc[0], k)),
            pl.BlockSpec((D, F_BLK), lambda k, r_sc: (0, k)),
        ],
        out_specs=pl.BlockSpec((CH, D), lambda k, r_sc: (0, 0)),
        scratch_shapes=[pltpu.VMEM((CH, D), jnp.float32)],
    )
    return pl.pallas_call(
        _matmul_body,
        grid_spec=grid_spec,
        out_shape=jax.ShapeDtypeStruct((CH, D), jnp.bfloat16),
        compiler_params=pltpu.CompilerParams(
            vmem_limit_bytes=100 * 1024 * 1024,
        ),
    )(r.astype(jnp.int32)[None], dy, w)


def _ar_body(p_ref, out_ref, yc_ref, ag_ref, acc_ref,
             yar_send, yar_recvs, cw_send, ccw_send, cw_recvs, ccw_recvs):
    my_x = lax.axis_index("x")
    my_y = lax.axis_index("y")
    my_z = lax.axis_index("z")
    y_right = (my_y + 1) % N_Y

    r = jnp.where(my_x == 0, my_z, 7 - my_z)
    cw_x = jnp.where(my_x == 0,
                     jnp.where(my_z == 3, 1, 0),
                     jnp.where(my_z == 0, 0, 1))
    cw_z = jnp.where(my_x == 0,
                     jnp.where(my_z == 3, 3, my_z + 1),
                     jnp.where(my_z == 0, 0, my_z - 1))
    ccw_x = jnp.where(my_x == 0,
                      jnp.where(my_z == 0, 1, 0),
                      jnp.where(my_z == 3, 0, 1))
    ccw_z = jnp.where(my_x == 0,
                      jnp.where(my_z == 0, 0, my_z - 1),
                      jnp.where(my_z == 3, 3, my_z + 1))

    barrier = pltpu.get_barrier_semaphore()
    for dev in ((my_x, (my_y - 1) % N_Y, my_z), (my_x, y_right, my_z),
                (cw_x, my_y, cw_z), (ccw_x, my_y, ccw_z)):
        pl.semaphore_signal(
            barrier, inc=1, device_id=dev,
            device_id_type=pl.DeviceIdType.MESH,
        )
    pl.semaphore_wait(barrier, 4)

    def y_hop(half, h):
        return pltpu.make_async_remote_copy(
            src_ref=yc_ref.at[half, h],
            dst_ref=yc_ref.at[half, h + 1],
            send_sem=yar_send.at[half],
            recv_sem=yar_recvs.at[half, h],
            device_id=(my_x, y_right, my_z),
            device_id_type=pl.DeviceIdType.MESH,
        )

    def cw_step(half, s):
        j = (r - s) % N_RING
        cols = pl.ds(half * HD, HD)
        return pltpu.make_async_remote_copy(
            src_ref=ag_ref.at[j, :, cols],
            dst_ref=ag_ref.at[j, :, cols],
            send_sem=cw_send.at[half],
            recv_sem=cw_recvs.at[half, s],
            device_id=(cw_x, my_y, cw_z),
            device_id_type=pl.DeviceIdType.MESH,
        )

    def ccw_step(half, s):
        j = (r + s) % N_RING
        cols = pl.ds(half * HD, HD)
        return pltpu.make_async_remote_copy(
            src_ref=ag_ref.at[j, :, cols],
            dst_ref=ag_ref.at[j, :, cols],
            send_sem=ccw_send.at[half],
            recv_sem=ccw_recvs.at[half, s],
            device_id=(ccw_x, my_y, ccw_z),
            device_id_type=pl.DeviceIdType.MESH,
        )

    def store(half, j):
        cols = pl.ds(half * HD, HD)
        out_ref[pl.ds(j * CH, CH), cols] = ag_ref[j, :, cols].astype(jnp.float32)

    acc_ref[...] = p_ref[...].astype(jnp.float32)
    yc_ref[0, 0] = p_ref[:, :HD]
    yc_ref[1, 0] = p_ref[:, HD:]

    rd = y_hop(0, 0)
    rd.start()
    rd.wait()
    rd = y_hop(0, 1)
    rd.start()
    acc_ref[:, :HD] += yc_ref[0, 1].astype(jnp.float32)
    rd.wait()
    rd = y_hop(0, 2)
    rd.start()
    acc_ref[:, :HD] += yc_ref[0, 2].astype(jnp.float32)
    rd.wait()
    acc_ref[:, :HD] += yc_ref[0, 3].astype(jnp.float32)
    ag_ref[r, :, :HD] = acc_ref[:, :HD].astype(jnp.bfloat16)
    out_ref[pl.ds(r * CH, CH), :HD] = acc_ref[:, :HD]

    for k in range(3):
        a = cw_step(0, k)
        a.start()
        b = ccw_step(0, k)
        b.start()
        y = y_hop(1, k)
        y.start()
        a.wait()
        b.wait()
        y.wait()
        store(0, (r - 1 - k) % N_RING)
        store(0, (r + 1 + k) % N_RING)
        if k < 2:
            acc_ref[:, HD:] += yc_ref[1, k + 1].astype(jnp.float32)

    acc_ref[:, HD:] += yc_ref[1, 3].astype(jnp.float32)
    ag_ref[r, :, HD:] = acc_ref[:, HD:].astype(jnp.bfloat16)
    out_ref[pl.ds(r * CH, CH), HD:] = acc_ref[:, HD:]

    for m in range(4):
        a = cw_step(1, m)
        a.start()
        b = None
        if m < 3:
            b = ccw_step(1, m)
            b.start()
        tail = None
        if m == 0:
            tail = cw_step(0, 3)
            tail.start()
        a.wait()
        if b is not None:
            b.wait()
        if tail is not None:
            tail.wait()
            store(0, (r - 4) % N_RING)
        store(1, (r - 1 - m) % N_RING)
        if m < 3:
            store(1, (r + 1 + m) % N_RING)


def _allreduce_allgather(p):
    return pl.pallas_call(
        _ar_body,
        out_shape=jax.ShapeDtypeStruct((M, D), jnp.float32),
        in_specs=[pl.BlockSpec(memory_space=pltpu.VMEM)],
        out_specs=pl.BlockSpec(memory_space=pltpu.VMEM),
        scratch_shapes=[
            pltpu.VMEM((2, N_Y, CH, HD), jnp.bfloat16),
            pltpu.VMEM((N_RING, CH, D), jnp.bfloat16),
            pltpu.VMEM((CH, D), jnp.float32),
            pltpu.SemaphoreType.DMA((2,)),
            pltpu.SemaphoreType.DMA((2, N_Y - 1)),
            pltpu.SemaphoreType.DMA((2,)),
            pltpu.SemaphoreType.DMA((2,)),
            pltpu.SemaphoreType.DMA((2, 4)),
            pltpu.SemaphoreType.DMA((2, 3)),
        ],
        compiler_params=pltpu.CompilerParams(
            collective_id=0,
            vmem_limit_bytes=100 * 1024 * 1024,
        ),
    )(p)


def kernel(dy, W):
    my_x = lax.axis_index("x")
    my_z = lax.axis_index("z")
    r = jnp.where(my_x == 0, my_z, 7 - my_z)
    p = _partial_matmul(r, dy, W)
    return _allreduce_allgather(p)


# device time: 136158 ns/iter; 2.8870x vs baseline; 1.0100x over previous
import jax
import jax.numpy as jnp
from jax import lax
from jax.experimental import pallas as pl
from jax.experimental.pallas import tpu as pltpu

M = 2048
D = 2048
HD = D // 2
F_SHARD = 8192
F_BLK = 512
N_Y = 4
N_RING = 8
CH = M // N_RING


def _matmul_body(r_ref, dy_ref, w_ref, out_ref, acc_ref):
    k = pl.program_id(0)

    @pl.when(k == 0)
    def _():
        acc_ref[...] = jnp.zeros_like(acc_ref)

    a = dy_ref[...].astype(jnp.bfloat16)
    b = w_ref[...].astype(jnp.bfloat16)
    acc_ref[...] += lax.dot_general(
        a, b, (((1,), (1,)), ((), ())), preferred_element_type=jnp.float32
    )

    @pl.when(k == pl.num_programs(0) - 1)
    def _():
        out_ref[...] = acc_ref[...].astype(jnp.bfloat16)


def _partial_matmul(r, dy, w):
    grid_spec = pltpu.PrefetchScalarGridSpec(
        num_scalar_prefetch=1,
        grid=(F_SHARD // F_BLK,),
        in_specs=[
            pl.BlockSpec((CH, F_BLK), lambda k, r_sc: (r_sc[0], k)),
            pl.BlockSpec((D, F_BLK), lambda k, r_sc: (0, k)),
        ],
        out_specs=pl.BlockSpec((CH, D), lambda k, r_sc: (0, 0)),
        scratch_shapes=[pltpu.VMEM((CH, D), jnp.float32)],
    )
    return pl.pallas_call(
        _matmul_body,
        grid_spec=grid_spec,
        out_shape=jax.ShapeDtypeStruct((CH, D), jnp.bfloat16),
        compiler_params=pltpu.CompilerParams(
            vmem_limit_bytes=100 * 1024 * 1024,
        ),
    )(r.astype(jnp.int32)[None], dy, w)


def _ar_body(p_ref, out_ref, yc_ref, ag_ref, acc_ref,
             yar_send, yar_recvs, cw_send, ccw_send, cw_recvs, ccw_recvs):
    my_x = lax.axis_index("x")
    my_y = lax.axis_index("y")
    my_z = lax.axis_index("z")
    y_right = (my_y + 1) % N_Y

    r = jnp.where(my_x == 0, my_z, 7 - my_z)
    cw_x = jnp.where(my_x == 0,
                     jnp.where(my_z == 3, 1, 0),
                     jnp.where(my_z == 0, 0, 1))
    cw_z = jnp.where(my_x == 0,
                     jnp.where(my_z == 3, 3, my_z + 1),
                     jnp.where(my_z == 0, 0, my_z - 1))
    ccw_x = jnp.where(my_x == 0,
                      jnp.where(my_z == 0, 1, 0),
                      jnp.where(my_z == 3, 0, 1))
    ccw_z = jnp.where(my_x == 0,
                      jnp.where(my_z == 0, 0, my_z - 1),
                      jnp.where(my_z == 3, 3, my_z + 1))

    barrier = pltpu.get_barrier_semaphore()
    for dev in ((my_x, (my_y - 1) % N_Y, my_z), (my_x, y_right, my_z),
                (cw_x, my_y, cw_z), (ccw_x, my_y, ccw_z)):
        pl.semaphore_signal(
            barrier, inc=1, device_id=dev,
            device_id_type=pl.DeviceIdType.MESH,
        )
    pl.semaphore_wait(barrier, 4)

    def y_hop(half, h):
        return pltpu.make_async_remote_copy(
            src_ref=yc_ref.at[half, h],
            dst_ref=yc_ref.at[half, h + 1],
            send_sem=yar_send.at[half],
            recv_sem=yar_recvs.at[half, h],
            device_id=(my_x, y_right, my_z),
            device_id_type=pl.DeviceIdType.MESH,
        )

    def cw_step(half, s):
        j = (r - s) % N_RING
        cols = pl.ds(half * HD, HD)
        return pltpu.make_async_remote_copy(
            src_ref=ag_ref.at[j, :, cols],
            dst_ref=ag_ref.at[j, :, cols],
            send_sem=cw_send.at[half],
            recv_sem=cw_recvs.at[half, s],
            device_id=(cw_x, my_y, cw_z),
            device_id_type=pl.DeviceIdType.MESH,
        )

    def ccw_step(half, s):
        j = (r + s) % N_RING
        cols = pl.ds(half * HD, HD)
        return pltpu.make_async_remote_copy(
            src_ref=ag_ref.at[j, :, cols],
            dst_ref=ag_ref.at[j, :, cols],
            send_sem=ccw_send.at[half],
            recv_sem=ccw_recvs.at[half, s],
            device_id=(ccw_x, my_y, ccw_z),
            device_id_type=pl.DeviceIdType.MESH,
        )

    def store(half, j):
        cols = pl.ds(half * HD, HD)
        out_ref[pl.ds(j * CH, CH), cols] = ag_ref[j, :, cols].astype(jnp.float32)

    acc_ref[...] = p_ref[...].astype(jnp.float32)
    yc_ref[0, 0] = p_ref[:, :HD]
    yc_ref[1, 0] = p_ref[:, HD:]

    rd = y_hop(0, 0)
    rd.start()
    rd.wait()
    rd = y_hop(0, 1)
    rd.start()
    acc_ref[:, :HD] += yc_ref[0, 1].astype(jnp.float32)
    rd.wait()
    rd = y_hop(0, 2)
    rd.start()
    acc_ref[:, :HD] += yc_ref[0, 2].astype(jnp.float32)
    rd.wait()
    acc_ref[:, :HD] += yc_ref[0, 3].astype(jnp.float32)
    ag_ref[r, :, :HD] = acc_ref[:, :HD].astype(jnp.bfloat16)
    out_ref[pl.ds(r * CH, CH), :HD] = acc_ref[:, :HD]


    rds = [cw_step(0, 0), ccw_step(0, 0), y_hop(1, 0)]
    for x in rds:
        x.start()
    for x in rds:
        x.wait()
    rds = [cw_step(0, 1), ccw_step(0, 1), y_hop(1, 1)]
    for x in rds:
        x.start()
    store(0, (r - 1) % N_RING)
    store(0, (r + 1) % N_RING)
    acc_ref[:, HD:] += yc_ref[1, 1].astype(jnp.float32)
    for x in rds:
        x.wait()
    rds = [cw_step(0, 2), ccw_step(0, 2), y_hop(1, 2)]
    for x in rds:
        x.start()
    store(0, (r - 2) % N_RING)
    store(0, (r + 2) % N_RING)
    acc_ref[:, HD:] += yc_ref[1, 2].astype(jnp.float32)
    for x in rds:
        x.wait()
    acc_ref[:, HD:] += yc_ref[1, 3].astype(jnp.float32)
    ag_ref[r, :, HD:] = acc_ref[:, HD:].astype(jnp.bfloat16)
    out_ref[pl.ds(r * CH, CH), HD:] = acc_ref[:, HD:]
    rds = [cw_step(1, 0), ccw_step(1, 0), cw_step(0, 3)]
    for x in rds:
        x.start()
    store(0, (r - 3) % N_RING)
    store(0, (r + 3) % N_RING)
    for x in rds:
        x.wait()
    rds = [cw_step(1, 1), ccw_step(1, 1)]
    for x in rds:
        x.start()
    store(0, (r - 4) % N_RING)
    store(1, (r - 1) % N_RING)
    store(1, (r + 1) % N_RING)
    for x in rds:
        x.wait()
    rds = [cw_step(1, 2), ccw_step(1, 2)]
    for x in rds:
        x.start()
    store(1, (r - 2) % N_RING)
    store(1, (r + 2) % N_RING)
    for x in rds:
        x.wait()
    rds = [cw_step(1, 3)]
    for x in rds:
        x.start()
    store(1, (r - 3) % N_RING)
    store(1, (r + 3) % N_RING)
    for x in rds:
        x.wait()
    store(1, (r - 4) % N_RING)


def _allreduce_allgather(p):
    return pl.pallas_call(
        _ar_body,
        out_shape=jax.ShapeDtypeStruct((M, D), jnp.float32),
        in_specs=[pl.BlockSpec(memory_space=pltpu.VMEM)],
        out_specs=pl.BlockSpec(memory_space=pltpu.VMEM),
        scratch_shapes=[
            pltpu.VMEM((2, N_Y, CH, HD), jnp.bfloat16),
            pltpu.VMEM((N_RING, CH, D), jnp.bfloat16),
            pltpu.VMEM((CH, D), jnp.float32),
            pltpu.SemaphoreType.DMA((2,)),
            pltpu.SemaphoreType.DMA((2, N_Y - 1)),
            pltpu.SemaphoreType.DMA((2,)),
            pltpu.SemaphoreType.DMA((2,)),
            pltpu.SemaphoreType.DMA((2, 4)),
            pltpu.SemaphoreType.DMA((2, 4)),
        ],
        compiler_params=pltpu.CompilerParams(
            collective_id=0,
            vmem_limit_bytes=100 * 1024 * 1024,
        ),
    )(p)


def kernel(dy, W):
    my_x = lax.axis_index("x")
    my_z = lax.axis_index("z")
    r = jnp.where(my_x == 0, my_z, 7 - my_z)
    p = _partial_matmul(r, dy, W)
    return _allreduce_allgather(p)


# device time: 124692 ns/iter; 3.1525x vs baseline; 1.0920x over previous
import jax
import jax.numpy as jnp
from jax import lax
from jax.experimental import pallas as pl
from jax.experimental.pallas import tpu as pltpu

M = 2048
D = 2048
HD = D // 2
F_SHARD = 8192
FBK = 1024
KPH = F_SHARD // FBK
N_Y = 4
N_RING = 8
CH = M // N_RING


def _body(dy_hbm, w_hbm, out_ref, dyv_ref, dyb_ref, wv_ref, acc_ref,
          yc_ref, ag_ref, dy_sem, w_sems,
          yar_send, yar_recvs, cw_send, ccw_send, cw_recvs, ccw_recvs):
    my_x = lax.axis_index("x")
    my_y = lax.axis_index("y")
    my_z = lax.axis_index("z")
    y_right = (my_y + 1) % N_Y

    r = jnp.where(my_x == 0, my_z, 7 - my_z)
    cw_x = jnp.where(my_x == 0,
                     jnp.where(my_z == 3, 1, 0),
                     jnp.where(my_z == 0, 0, 1))
    cw_z = jnp.where(my_x == 0,
                     jnp.where(my_z == 3, 3, my_z + 1),
                     jnp.where(my_z == 0, 0, my_z - 1))
    ccw_x = jnp.where(my_x == 0,
                      jnp.where(my_z == 0, 1, 0),
                      jnp.where(my_z == 3, 0, 1))
    ccw_z = jnp.where(my_x == 0,
                      jnp.where(my_z == 0, 0, my_z - 1),
                      jnp.where(my_z == 3, 3, my_z + 1))

    def w_dma(i):
        h, k = divmod(i, KPH)
        return pltpu.make_async_copy(
            w_hbm.at[pl.ds(h * HD, HD), pl.ds(k * FBK, FBK)],
            wv_ref.at[i % 2],
            w_sems.at[i % 2],
        )

    dy_dma = pltpu.make_async_copy(
        dy_hbm.at[pl.ds(r * CH, CH), :], dyv_ref, dy_sem)
    dy_dma.start()
    w_dma(0).start()

    barrier = pltpu.get_barrier_semaphore()
    for dev in ((my_x, (my_y - 1) % N_Y, my_z), (my_x, y_right, my_z),
                (cw_x, my_y, cw_z), (ccw_x, my_y, ccw_z)):
        pl.semaphore_signal(
            barrier, inc=1, device_id=dev,
            device_id_type=pl.DeviceIdType.MESH,
        )
    pl.semaphore_wait(barrier, 4)

    dy_dma.wait()
    dyb_ref[...] = dyv_ref[...].astype(jnp.bfloat16)

    def dot_block(i):
        h, k = divmod(i, KPH)
        w_dma(i).wait()
        if i + 1 < 2 * KPH:
            w_dma(i + 1).start()
        a = dyb_ref[:, k * FBK:(k + 1) * FBK]
        b = wv_ref[i % 2].astype(jnp.bfloat16)
        cols = slice(0, HD) if h == 0 else slice(HD, D)
        acc_ref[:, cols] += lax.dot_general(
            a, b, (((1,), (1,)), ((), ())),
            preferred_element_type=jnp.float32,
        )

    acc_ref[...] = jnp.zeros_like(acc_ref)
    for i in range(KPH):
        dot_block(i)
    yc_ref[0, 0] = acc_ref[:, :HD].astype(jnp.bfloat16)

    def y_hop(half, h):
        return pltpu.make_async_remote_copy(
            src_ref=yc_ref.at[half, h],
            dst_ref=yc_ref.at[half, h + 1],
            send_sem=yar_send.at[half],
            recv_sem=yar_recvs.at[half, h],
            device_id=(my_x, y_right, my_z),
            device_id_type=pl.DeviceIdType.MESH,
        )

    def cw_step(half, s):
        j = (r - s) % N_RING
        cols = pl.ds(half * HD, HD)
        return pltpu.make_async_remote_copy(
            src_ref=ag_ref.at[j, :, cols],
            dst_ref=ag_ref.at[j, :, cols],
            send_sem=cw_send.at[half],
            recv_sem=cw_recvs.at[half, s],
            device_id=(cw_x, my_y, cw_z),
            device_id_type=pl.DeviceIdType.MESH,
        )

    def ccw_step(half, s):
        j = (r + s) % N_RING
        cols = pl.ds(half * HD, HD)
        return pltpu.make_async_remote_copy(
            src_ref=ag_ref.at[j, :, cols],
            dst_ref=ag_ref.at[j, :, cols],
            send_sem=ccw_send.at[half],
            recv_sem=ccw_recvs.at[half, s],
            device_id=(ccw_x, my_y, ccw_z),
            device_id_type=pl.DeviceIdType.MESH,
        )

    def store(half, j):
        cols = pl.ds(half * HD, HD)
        out_ref[pl.ds(j * CH, CH), cols] = ag_ref[j, :, cols].astype(jnp.float32)

    rd = y_hop(0, 0)
    rd.start()
    for i in range(KPH, KPH + 3):
        dot_block(i)
    rd.wait()
    rd = y_hop(0, 1)
    rd.start()
    for i in range(KPH + 3, KPH + 6):
        dot_block(i)
    acc_ref[:, :HD] += yc_ref[0, 1].astype(jnp.float32)
    rd.wait()
    rd = y_hop(0, 2)
    rd.start()
    for i in range(KPH + 6, 2 * KPH):
        dot_block(i)
    acc_ref[:, :HD] += yc_ref[0, 2].astype(jnp.float32)
    rd.wait()
    acc_ref[:, :HD] += yc_ref[0, 3].astype(jnp.float32)
    ag_ref[r, :, :HD] = acc_ref[:, :HD].astype(jnp.bfloat16)
    out_ref[pl.ds(r * CH, CH), :HD] = acc_ref[:, :HD]
    yc_ref[1, 0] = acc_ref[:, HD:].astype(jnp.bfloat16)

    rds = [cw_step(0, 0), ccw_step(0, 0), y_hop(1, 0)]
    for x in rds:
        x.start()
    for x in rds:
        x.wait()
    rds = [cw_step(0, 1), ccw_step(0, 1), y_hop(1, 1)]
    for x in rds:
        x.start()
    store(0, (r - 1) % N_RING)
    store(0, (r + 1) % N_RING)
    acc_ref[:, HD:] += yc_ref[1, 1].astype(jnp.float32)
    for x in rds:
        x.wait()
    rds = [cw_step(0, 2), ccw_step(0, 2), y_hop(1, 2)]
    for x in rds:
        x.start()
    store(0, (r - 2) % N_RING)
    store(0, (r + 2) % N_RING)
    acc_ref[:, HD:] += yc_ref[1, 2].astype(jnp.float32)
    for x in rds:
        x.wait()
    acc_ref[:, HD:] += yc_ref[1, 3].astype(jnp.float32)
    ag_ref[r, :, HD:] = acc_ref[:, HD:].astype(jnp.bfloat16)
    out_ref[pl.ds(r * CH, CH), HD:] = acc_ref[:, HD:]
    rds = [cw_step(1, 0), ccw_step(1, 0), cw_step(0, 3)]
    for x in rds:
        x.start()
    store(0, (r - 3) % N_RING)
    store(0, (r + 3) % N_RING)
    for x in rds:
        x.wait()
    rds = [cw_step(1, 1), ccw_step(1, 1)]
    for x in rds:
        x.start()
    store(0, (r - 4) % N_RING)
    store(1, (r - 1) % N_RING)
    store(1, (r + 1) % N_RING)
    for x in rds:
        x.wait()
    rds = [cw_step(1, 2), ccw_step(1, 2)]
    for x in rds:
        x.start()
    store(1, (r - 2) % N_RING)
    store(1, (r + 2) % N_RING)
    for x in rds:
        x.wait()
    rds = [cw_step(1, 3)]
    for x in rds:
        x.start()
    store(1, (r - 3) % N_RING)
    store(1, (r + 3) % N_RING)
    for x in rds:
        x.wait()
    store(1, (r - 4) % N_RING)


def kernel(dy, W):
    return pl.pallas_call(
        _body,
        out_shape=jax.ShapeDtypeStruct((M, D), jnp.float32),
        in_specs=[
            pl.BlockSpec(memory_space=pltpu.MemorySpace.HBM),
            pl.BlockSpec(memory_space=pltpu.MemorySpace.HBM),
        ],
        out_specs=pl.BlockSpec(memory_space=pltpu.VMEM),
        scratch_shapes=[
            pltpu.VMEM((CH, F_SHARD), jnp.float32),
            pltpu.VMEM((CH, F_SHARD), jnp.bfloat16),
            pltpu.VMEM((2, HD, FBK), jnp.float32),
            pltpu.VMEM((CH, D), jnp.float32),
            pltpu.VMEM((2, N_Y, CH, HD), jnp.bfloat16),
            pltpu.VMEM((N_RING, CH, D), jnp.bfloat16),
            pltpu.SemaphoreType.DMA,
            pltpu.SemaphoreType.DMA((2,)),
            pltpu.SemaphoreType.DMA((2,)),
            pltpu.SemaphoreType.DMA((2, N_Y - 1)),
            pltpu.SemaphoreType.DMA((2,)),
            pltpu.SemaphoreType.DMA((2,)),
            pltpu.SemaphoreType.DMA((2, 4)),
            pltpu.SemaphoreType.DMA((2, 4)),
        ],
        compiler_params=pltpu.CompilerParams(
            collective_id=0,
            vmem_limit_bytes=110 * 1024 * 1024,
        ),
    )(dy, W)


# device time: 120246 ns/iter; 3.2691x vs baseline; 1.0370x over previous
import jax
import jax.numpy as jnp
from jax import lax
from jax.experimental import pallas as pl
from jax.experimental.pallas import tpu as pltpu

M = 2048
D = 2048
HD = D // 2
F_SHARD = 8192
FBK = 1024
KPH = F_SHARD // FBK
N_Y = 4
N_RING = 8
CH = M // N_RING


def _body(dy_hbm, w_hbm, out_ref, dyv_ref, dyb_ref, wv_ref, acc_ref,
          yc_ref, ag_ref, st_ref, dy_sem, w_sems,
          yar_send, yar_recvs, cw_send, ccw_send, cw_recvs, ccw_recvs,
          st_sems, own_sems):
    my_x = lax.axis_index("x")
    my_y = lax.axis_index("y")
    my_z = lax.axis_index("z")
    y_right = (my_y + 1) % N_Y

    r = jnp.where(my_x == 0, my_z, 7 - my_z)
    cw_x = jnp.where(my_x == 0,
                     jnp.where(my_z == 3, 1, 0),
                     jnp.where(my_z == 0, 0, 1))
    cw_z = jnp.where(my_x == 0,
                     jnp.where(my_z == 3, 3, my_z + 1),
                     jnp.where(my_z == 0, 0, my_z - 1))
    ccw_x = jnp.where(my_x == 0,
                      jnp.where(my_z == 0, 1, 0),
                      jnp.where(my_z == 3, 0, 1))
    ccw_z = jnp.where(my_x == 0,
                      jnp.where(my_z == 0, 0, my_z - 1),
                      jnp.where(my_z == 3, 3, my_z + 1))

    def w_dma(i):
        h, k = divmod(i, KPH)
        return pltpu.make_async_copy(
            w_hbm.at[pl.ds(h * HD, HD), pl.ds(k * FBK, FBK)],
            wv_ref.at[i % 2],
            w_sems.at[i % 2],
        )

    dy_dma = pltpu.make_async_copy(
        dy_hbm.at[pl.ds(r * CH, CH), :], dyv_ref, dy_sem)
    dy_dma.start()
    w_dma(0).start()

    barrier = pltpu.get_barrier_semaphore()
    for dev in ((my_x, (my_y - 1) % N_Y, my_z), (my_x, y_right, my_z),
                (cw_x, my_y, cw_z), (ccw_x, my_y, ccw_z)):
        pl.semaphore_signal(
            barrier, inc=1, device_id=dev,
            device_id_type=pl.DeviceIdType.MESH,
        )
    pl.semaphore_wait(barrier, 4)

    dy_dma.wait()
    dyb_ref[...] = dyv_ref[...].astype(jnp.bfloat16)

    def dot_block(i):
        h, k = divmod(i, KPH)
        w_dma(i).wait()
        if i + 1 < 2 * KPH:
            w_dma(i + 1).start()
        a = dyb_ref[:, k * FBK:(k + 1) * FBK]
        b = wv_ref[i % 2].astype(jnp.bfloat16)
        cols = slice(0, HD) if h == 0 else slice(HD, D)
        acc_ref[:, cols] += lax.dot_general(
            a, b, (((1,), (1,)), ((), ())),
            preferred_element_type=jnp.float32,
        )

    acc_ref[...] = jnp.zeros_like(acc_ref)
    for i in range(KPH):
        dot_block(i)
    yc_ref[0, 0] = acc_ref[:, :HD].astype(jnp.bfloat16)

    def y_hop(half, h):
        return pltpu.make_async_remote_copy(
            src_ref=yc_ref.at[half, h],
            dst_ref=yc_ref.at[half, h + 1],
            send_sem=yar_send.at[half],
            recv_sem=yar_recvs.at[half, h],
            device_id=(my_x, y_right, my_z),
            device_id_type=pl.DeviceIdType.MESH,
        )

    def cw_step(half, s):
        j = (r - s) % N_RING
        cols = pl.ds(half * HD, HD)
        return pltpu.make_async_remote_copy(
            src_ref=ag_ref.at[j, :, cols],
            dst_ref=ag_ref.at[j, :, cols],
            send_sem=cw_send.at[half],
            recv_sem=cw_recvs.at[half, s],
            device_id=(cw_x, my_y, cw_z),
            device_id_type=pl.DeviceIdType.MESH,
        )

    def ccw_step(half, s):
        j = (r + s) % N_RING
        cols = pl.ds(half * HD, HD)
        return pltpu.make_async_remote_copy(
            src_ref=ag_ref.at[j, :, cols],
            dst_ref=ag_ref.at[j, :, cols],
            send_sem=ccw_send.at[half],
            recv_sem=ccw_recvs.at[half, s],
            device_id=(ccw_x, my_y, ccw_z),
            device_id_type=pl.DeviceIdType.MESH,
        )

    st_state = {0: None, 1: None}

    def store(half, j):
        cols = pl.ds(half * HD, HD)
        slot = store.counter % 2
        store.counter += 1
        if st_state[slot] is not None:
            st_state[slot].wait()
        st_ref[slot] = ag_ref[j, :, cols].astype(jnp.float32)
        dma = pltpu.make_async_copy(
            st_ref.at[slot],
            out_ref.at[pl.ds(j * CH, CH), cols],
            st_sems.at[slot],
        )
        dma.start()
        st_state[slot] = dma

    store.counter = 0

    def store_own(half):
        cols = pl.ds(half * HD, HD)
        dma = pltpu.make_async_copy(
            acc_ref.at[:, cols],
            out_ref.at[pl.ds(r * CH, CH), cols],
            own_sems.at[half],
        )
        dma.start()
        return dma

    rd = y_hop(0, 0)
    rd.start()
    for i in range(KPH, KPH + 3):
        dot_block(i)
    rd.wait()
    rd = y_hop(0, 1)
    rd.start()
    for i in range(KPH + 3, KPH + 6):
        dot_block(i)
    acc_ref[:, :HD] += yc_ref[0, 1].astype(jnp.float32)
    rd.wait()
    rd = y_hop(0, 2)
    rd.start()
    for i in range(KPH + 6, 2 * KPH):
        dot_block(i)
    acc_ref[:, :HD] += yc_ref[0, 2].astype(jnp.float32)
    rd.wait()
    acc_ref[:, :HD] += yc_ref[0, 3].astype(jnp.float32)
    ag_ref[r, :, :HD] = acc_ref[:, :HD].astype(jnp.bfloat16)
    own0 = store_own(0)
    yc_ref[1, 0] = acc_ref[:, HD:].astype(jnp.bfloat16)

    rds = [cw_step(0, 0), ccw_step(0, 0), y_hop(1, 0)]
    for x in rds:
        x.start()
    for x in rds:
        x.wait()
    rds = [cw_step(0, 1), ccw_step(0, 1), y_hop(1, 1)]
    for x in rds:
        x.start()
    store(0, (r - 1) % N_RING)
    store(0, (r + 1) % N_RING)
    acc_ref[:, HD:] += yc_ref[1, 1].astype(jnp.float32)
    for x in rds:
        x.wait()
    rds = [cw_step(0, 2), ccw_step(0, 2), y_hop(1, 2)]
    for x in rds:
        x.start()
    store(0, (r - 2) % N_RING)
    store(0, (r + 2) % N_RING)
    acc_ref[:, HD:] += yc_ref[1, 2].astype(jnp.float32)
    for x in rds:
        x.wait()
    acc_ref[:, HD:] += yc_ref[1, 3].astype(jnp.float32)
    ag_ref[r, :, HD:] = acc_ref[:, HD:].astype(jnp.bfloat16)
    own1 = store_own(1)
    rds = [cw_step(1, 0), ccw_step(1, 0), cw_step(0, 3)]
    for x in rds:
        x.start()
    store(0, (r - 3) % N_RING)
    store(0, (r + 3) % N_RING)
    for x in rds:
        x.wait()
    rds = [cw_step(1, 1), ccw_step(1, 1)]
    for x in rds:
        x.start()
    store(0, (r - 4) % N_RING)
    store(1, (r - 1) % N_RING)
    store(1, (r + 1) % N_RING)
    for x in rds:
        x.wait()
    rds = [cw_step(1, 2), ccw_step(1, 2)]
    for x in rds:
        x.start()
    store(1, (r - 2) % N_RING)
    store(1, (r + 2) % N_RING)
    for x in rds:
        x.wait()
    rds = [cw_step(1, 3)]
    for x in rds:
        x.start()
    store(1, (r - 3) % N_RING)
    store(1, (r + 3) % N_RING)
    for x in rds:
        x.wait()
    store(1, (r - 4) % N_RING)

    own0.wait()
    own1.wait()
    for slot in (0, 1):
        if st_state[slot] is not None:
            st_state[slot].wait()


def kernel(dy, W):
    return pl.pallas_call(
        _body,
        out_shape=jax.ShapeDtypeStruct((M, D), jnp.float32),
        in_specs=[
            pl.BlockSpec(memory_space=pltpu.MemorySpace.HBM),
            pl.BlockSpec(memory_space=pltpu.MemorySpace.HBM),
        ],
        out_specs=pl.BlockSpec(memory_space=pltpu.MemorySpace.HBM),
        scratch_shapes=[
            pltpu.VMEM((CH, F_SHARD), jnp.float32),
            pltpu.VMEM((CH, F_SHARD), jnp.bfloat16),
            pltpu.VMEM((2, HD, FBK), jnp.float32),
            pltpu.VMEM((CH, D), jnp.float32),
            pltpu.VMEM((2, N_Y, CH, HD), jnp.bfloat16),
            pltpu.VMEM((N_RING, CH, D), jnp.bfloat16),
            pltpu.VMEM((2, CH, HD), jnp.float32),
            pltpu.SemaphoreType.DMA,
            pltpu.SemaphoreType.DMA((2,)),
            pltpu.SemaphoreType.DMA((2,)),
            pltpu.SemaphoreType.DMA((2, N_Y - 1)),
            pltpu.SemaphoreType.DMA((2,)),
            pltpu.SemaphoreType.DMA((2,)),
            pltpu.SemaphoreType.DMA((2, 4)),
            pltpu.SemaphoreType.DMA((2, 4)),
            pltpu.SemaphoreType.DMA((2,)),
            pltpu.SemaphoreType.DMA((2,)),
        ],
        compiler_params=pltpu.CompilerParams(
            collective_id=0,
            vmem_limit_bytes=110 * 1024 * 1024,
        ),
    )(dy, W)
